# baseline (speedup 1.0000x reference)
import jax
import jax.numpy as jnp
from jax.experimental import pallas as pl
from jax.experimental.pallas import tpu as pltpu

_LO = -1.03
_HI = 1.03

_BLOCK = 4096


def _mlp_kernel(x_ref, w1_ref, b1_ref, w2_ref, b2_ref, o_ref):
    x = x_ref[...]
    pts = x[:, :3]
    mask_out = ((pts < _LO) | (pts > _HI)).any(axis=1)
    xb = x.astype(jnp.bfloat16)
    h = jax.lax.dot_general(xb, w1_ref[...], (((1,), (0,)), ((), ())),
                            preferred_element_type=jnp.float32)
    h = jnp.maximum(h + b1_ref[...], 0.0)
    v = jax.lax.dot_general(h.astype(jnp.bfloat16), w2_ref[...],
                            (((1,), (0,)), ((), ())),
                            preferred_element_type=jnp.float32)
    v = v + b2_ref[...]
    o_ref[...] = jnp.where(mask_out[:, None], 0.0, v)


def kernel(xt, W1, b1, W2, b2):
    n = xt.shape[0]
    grid = (n // _BLOCK,)
    return pl.pallas_call(
        _mlp_kernel,
        grid=grid,
        in_specs=[
            pl.BlockSpec((_BLOCK, 4), lambda i: (i, 0)),
            pl.BlockSpec((4, 64), lambda i: (0, 0)),
            pl.BlockSpec((1, 64), lambda i: (0, 0)),
            pl.BlockSpec((64, 3), lambda i: (0, 0)),
            pl.BlockSpec((1, 3), lambda i: (0, 0)),
        ],
        out_specs=pl.BlockSpec((_BLOCK, 3), lambda i: (i, 0)),
        out_shape=jax.ShapeDtypeStruct((n, 3), xt.dtype),
        compiler_params=pltpu.CompilerParams(
            dimension_semantics=("arbitrary",),
        ),
    )(xt, W1.astype(jnp.bfloat16), b1.reshape(1, 64),
      W2.astype(jnp.bfloat16), b2.reshape(1, 3))
